# baseline (speedup 1.0000x reference)
import functools
import math

import jax
import jax.numpy as jnp
from jax import lax
from jax.experimental import pallas as pl
from jax.experimental.pallas import tpu as pltpu
from jax.experimental.pallas import tpu_sc as plsc

E = 8
TOPK = 2
D = 2048
DFF = 1408
DSH = 5632
T = 2048

FINAL_SCALE = 1.0 / math.sqrt(TOPK)

BLK = 128
S = T * TOPK + E * BLK
NBLK = S // BLK

NC = 2
NSUB = 16
NW = NC * NSUB
TPW = T // NW
RND = 32
CHUNK = 16


def _router_body(x_ref, rw_ref, sgw1_ref, w1_ref, w2_ref, p1_ref, p2_ref,
                 bex_ref, gate_ref):
    x = x_ref[...]
    logits = jnp.dot(x, rw_ref[...], preferred_element_type=jnp.float32)
    col = lax.broadcasted_iota(jnp.int32, (T, E), 1)
    m1 = jnp.max(logits, axis=-1, keepdims=True)
    i1 = jnp.min(jnp.where(logits == m1, col, E), axis=-1, keepdims=True)
    oh1 = col == i1
    l2 = jnp.where(oh1, -jnp.inf, logits)
    m2 = jnp.max(l2, axis=-1, keepdims=True)
    i2 = jnp.min(jnp.where(l2 == m2, col, E), axis=-1, keepdims=True)
    oh2 = col == i2
    w1 = 1.0 / (1.0 + jnp.exp(m2 - m1))
    w1_ref[...] = w1
    w2_ref[...] = 1.0 - w1

    m = oh1.astype(jnp.float32) + oh2.astype(jnp.float32)
    counts = jnp.sum(m, axis=0, keepdims=True)
    counts_r = jnp.floor((counts + (BLK - 1)) / BLK) * BLK
    r8 = lax.broadcasted_iota(jnp.int32, (E, E), 0)
    c8 = lax.broadcasted_iota(jnp.int32, (E, E), 1)
    u8 = (r8 < c8).astype(jnp.float32)
    starts = jnp.dot(counts_r, u8, preferred_element_type=jnp.float32)

    rt = lax.broadcasted_iota(jnp.int32, (T, T), 0)
    ct = lax.broadcasted_iota(jnp.int32, (T, T), 1)
    ltri = (ct < rt).astype(jnp.float32)
    ranks = jnp.dot(ltri, m, preferred_element_type=jnp.float32)
    posv = starts + ranks
    p1_ref[...] = jnp.sum(jnp.where(oh1, posv, 0.0), axis=-1,
                          keepdims=True).astype(jnp.int32)
    p2_ref[...] = jnp.sum(jnp.where(oh2, posv, 0.0), axis=-1,
                          keepdims=True).astype(jnp.int32)

    i8 = (r8 == c8).astype(jnp.float32)
    starts_c = lax.dot_general(i8, starts, (((1,), (1,)), ((), ())),
                               preferred_element_type=jnp.float32)
    blk0 = (lax.broadcasted_iota(jnp.int32, (E, NBLK), 1)
            * BLK).astype(jnp.float32)
    ge = (blk0 >= starts_c).astype(jnp.int32)
    bex_ref[...] = jnp.sum(ge, axis=0, keepdims=True) - 1

    gate_ref[...] = jax.nn.sigmoid(
        jnp.dot(x, sgw1_ref[...], preferred_element_type=jnp.float32))


def _scatter_sc(x_hbm, p1_hbm, p2_hbm, sorted_hbm, rows_v, i1_v, i2_v, sem):
    wid = lax.axis_index("s") * NC + lax.axis_index("c")
    for r in range(TPW // RND):
        base = wid * TPW + r * RND
        pltpu.sync_copy(x_hbm.at[pl.ds(base, RND)], rows_v)
        pltpu.sync_copy(p1_hbm.at[wid, pl.ds(r * RND, RND)], i1_v)
        pltpu.sync_copy(p2_hbm.at[wid, pl.ds(r * RND, RND)], i2_v)
        pltpu.async_copy(rows_v, sorted_hbm.at[i1_v], sem).wait()
        pltpu.async_copy(rows_v, sorted_hbm.at[i2_v], sem).wait()


def _hup_body(bex_ref, xs_ref, gw_ref, uw_ref, h_ref):
    xt = xs_ref[...].astype(jnp.bfloat16)
    g = jnp.dot(xt, gw_ref[0], preferred_element_type=jnp.float32)
    u = jnp.dot(xt, uw_ref[0], preferred_element_type=jnp.float32)
    h_ref[...] = (g * jax.nn.sigmoid(g)) * u


def _down_body(bex_ref, h_ref, dw_ref, out_ref):
    out_ref[...] = jnp.dot(h_ref[...].astype(jnp.bfloat16), dw_ref[0],
                           preferred_element_type=jnp.float32)


def _gather_sc(dout_hbm, p1_hbm, p2_hbm, g1_hbm, g2_hbm,
               r1_v, r2_v, i1_v, i2_v, sem):
    wid = lax.axis_index("s") * NC + lax.axis_index("c")
    for c in range(TPW // CHUNK):
        base = wid * TPW + c * CHUNK
        pltpu.sync_copy(p1_hbm.at[wid, pl.ds(c * CHUNK, CHUNK)], i1_v)
        pltpu.sync_copy(p2_hbm.at[wid, pl.ds(c * CHUNK, CHUNK)], i2_v)
        cp1 = pltpu.async_copy(dout_hbm.at[i1_v], r1_v, sem)
        cp2 = pltpu.async_copy(dout_hbm.at[i2_v], r2_v, sem)
        cp1.wait()
        cp2.wait()
        pltpu.sync_copy(r1_v, g1_hbm.at[pl.ds(base, CHUNK)])
        pltpu.sync_copy(r2_v, g2_hbm.at[pl.ds(base, CHUNK)])


def _shared_body(x_ref, gw_ref, uw_ref, dw_ref, gate_ref, out_ref, *,
                 bt, ns):
    s = pl.program_id(0)
    t = pl.program_id(1)
    rows = pl.ds(t * bt, bt)
    xt = x_ref[...]
    g = jnp.dot(xt, gw_ref[...], preferred_element_type=jnp.float32)
    u = jnp.dot(xt, uw_ref[...], preferred_element_type=jnp.float32)
    h = (g * jax.nn.sigmoid(g)) * u
    part = jnp.dot(h, dw_ref[...], preferred_element_type=jnp.float32)

    @pl.when(s == 0)
    def _init():
        out_ref[rows, :] = part

    @pl.when(s != 0)
    def _acc():
        out_ref[rows, :] += part

    @pl.when(s == ns - 1)
    def _finalize():
        out_ref[rows, :] = gate_ref[...] * out_ref[rows, :]


def _final_body(g1_ref, g2_ref, shg_ref, w1_ref, w2_ref, out_ref):
    out_ref[...] = (w1_ref[...] * g1_ref[...] + w2_ref[...] * g2_ref[...]
                    + shg_ref[...]) * FINAL_SCALE


def kernel(hidden_states, expert_gate_w, expert_up_w, expert_down_w,
           shared_gate_w, shared_up_w, shared_down_w, router_w,
           shared_expert_gate_w):
    x = hidden_states.reshape(T, D)

    w1, w2, p1, p2, bex, gate = pl.pallas_call(
        _router_body,
        out_shape=(
            jax.ShapeDtypeStruct((T, 1), jnp.float32),
            jax.ShapeDtypeStruct((T, 1), jnp.float32),
            jax.ShapeDtypeStruct((T, 1), jnp.int32),
            jax.ShapeDtypeStruct((T, 1), jnp.int32),
            jax.ShapeDtypeStruct((1, NBLK), jnp.int32),
            jax.ShapeDtypeStruct((T, 1), jnp.float32),
        ),
    )(x, router_w, shared_expert_gate_w)

    p1w = p1.reshape(NW, TPW)
    p2w = p2.reshape(NW, TPW)
    bexf = bex.reshape(NBLK)

    BT = 512
    BS = 512
    NS = DSH // BS
    shg = pl.pallas_call(
        functools.partial(_shared_body, bt=BT, ns=NS),
        grid=(NS, T // BT),
        in_specs=[
            pl.BlockSpec((BT, D), lambda s, t: (t, 0)),
            pl.BlockSpec((D, BS), lambda s, t: (0, s)),
            pl.BlockSpec((D, BS), lambda s, t: (0, s)),
            pl.BlockSpec((BS, D), lambda s, t: (s, 0)),
            pl.BlockSpec((BT, 1), lambda s, t: (t, 0)),
        ],
        out_specs=pl.BlockSpec((T, D), lambda s, t: (0, 0)),
        out_shape=jax.ShapeDtypeStruct((T, D), jnp.float32),
    )(x, shared_gate_w, shared_up_w, shared_down_w, gate)

    mesh = plsc.VectorSubcoreMesh(core_axis_name="c", subcore_axis_name="s")
    sorted_x = pl.kernel(
        _scatter_sc,
        mesh=mesh,
        out_type=jax.ShapeDtypeStruct((S, D), jnp.float32),
        scratch_types=[
            pltpu.VMEM((RND, D), jnp.float32),
            pltpu.VMEM((RND,), jnp.int32),
            pltpu.VMEM((RND,), jnp.int32),
            pltpu.SemaphoreType.DMA,
        ],
    )(x, p1w, p2w)

    h = pl.pallas_call(
        _hup_body,
        grid_spec=pltpu.PrefetchScalarGridSpec(
            num_scalar_prefetch=1,
            grid=(NBLK,),
            in_specs=[
                pl.BlockSpec((BLK, D), lambda i, b: (i, 0)),
                pl.BlockSpec((1, D, DFF), lambda i, b: (b[i], 0, 0)),
                pl.BlockSpec((1, D, DFF), lambda i, b: (b[i], 0, 0)),
            ],
            out_specs=pl.BlockSpec((BLK, DFF), lambda i, b: (i, 0)),
        ),
        out_shape=jax.ShapeDtypeStruct((S, DFF), jnp.float32),
    )(bexf, sorted_x, expert_gate_w.astype(jnp.bfloat16),
      expert_up_w.astype(jnp.bfloat16))

    dout = pl.pallas_call(
        _down_body,
        grid_spec=pltpu.PrefetchScalarGridSpec(
            num_scalar_prefetch=1,
            grid=(NBLK,),
            in_specs=[
                pl.BlockSpec((BLK, DFF), lambda i, b: (i, 0)),
                pl.BlockSpec((1, DFF, D), lambda i, b: (b[i], 0, 0)),
            ],
            out_specs=pl.BlockSpec((BLK, D), lambda i, b: (i, 0)),
        ),
        out_shape=jax.ShapeDtypeStruct((S, D), jnp.float32),
    )(bexf, h, expert_down_w.astype(jnp.bfloat16))

    g1, g2 = pl.kernel(
        _gather_sc,
        mesh=mesh,
        out_type=(
            jax.ShapeDtypeStruct((T, D), jnp.float32),
            jax.ShapeDtypeStruct((T, D), jnp.float32),
        ),
        scratch_types=[
            pltpu.VMEM((CHUNK, D), jnp.float32),
            pltpu.VMEM((CHUNK, D), jnp.float32),
            pltpu.VMEM((CHUNK,), jnp.int32),
            pltpu.VMEM((CHUNK,), jnp.int32),
            pltpu.SemaphoreType.DMA,
        ],
    )(dout, p1w, p2w)

    out = pl.pallas_call(
        _final_body,
        grid=(T // BT,),
        in_specs=[
            pl.BlockSpec((BT, D), lambda t: (t, 0)),
            pl.BlockSpec((BT, D), lambda t: (t, 0)),
            pl.BlockSpec((BT, D), lambda t: (t, 0)),
            pl.BlockSpec((BT, 1), lambda t: (t, 0)),
            pl.BlockSpec((BT, 1), lambda t: (t, 0)),
        ],
        out_specs=pl.BlockSpec((BT, D), lambda t: (t, 0)),
        out_shape=jax.ShapeDtypeStruct((T, D), jnp.float32),
    )(g1, g2, shg, w1, w2)

    return out


# speedup vs baseline: 1.1578x; 1.1578x over previous
import functools
import math

import jax
import jax.numpy as jnp
from jax import lax
from jax.experimental import pallas as pl
from jax.experimental.pallas import tpu as pltpu
from jax.experimental.pallas import tpu_sc as plsc

E = 8
TOPK = 2
D = 2048
DFF = 1408
DSH = 5632
T = 2048

FINAL_SCALE = 1.0 / math.sqrt(TOPK)

BLK = 128
S = T * TOPK + E * BLK
NBLK = S // BLK

NC = 2
NSUB = 16
NW = NC * NSUB
TPW = T // NW
RND = 32
CHUNK = 16


def _router_body(x_ref, rw_ref, sgw1_ref, w1_ref, w2_ref, p1_ref, p2_ref,
                 bex_ref, gate_ref):
    x = x_ref[...]
    logits = jnp.dot(x, rw_ref[...], preferred_element_type=jnp.float32)
    col = lax.broadcasted_iota(jnp.int32, (T, E), 1)
    m1 = jnp.max(logits, axis=-1, keepdims=True)
    i1 = jnp.min(jnp.where(logits == m1, col, E), axis=-1, keepdims=True)
    oh1 = col == i1
    l2 = jnp.where(oh1, -jnp.inf, logits)
    m2 = jnp.max(l2, axis=-1, keepdims=True)
    i2 = jnp.min(jnp.where(l2 == m2, col, E), axis=-1, keepdims=True)
    oh2 = col == i2
    w1 = 1.0 / (1.0 + jnp.exp(m2 - m1))
    w1_ref[...] = w1
    w2_ref[...] = 1.0 - w1

    m = oh1.astype(jnp.float32) + oh2.astype(jnp.float32)
    counts = jnp.sum(m, axis=0, keepdims=True)
    counts_r = jnp.floor((counts + (BLK - 1)) / BLK) * BLK
    r8 = lax.broadcasted_iota(jnp.int32, (E, E), 0)
    c8 = lax.broadcasted_iota(jnp.int32, (E, E), 1)
    u8 = (r8 < c8).astype(jnp.float32)
    starts = jnp.dot(counts_r, u8, preferred_element_type=jnp.float32)

    rt = lax.broadcasted_iota(jnp.int32, (T, T), 0)
    ct = lax.broadcasted_iota(jnp.int32, (T, T), 1)
    ltri = (ct < rt).astype(jnp.float32)
    ranks = jnp.dot(ltri, m, preferred_element_type=jnp.float32)
    posv = starts + ranks
    p1_ref[...] = jnp.sum(jnp.where(oh1, posv, 0.0), axis=-1,
                          keepdims=True).astype(jnp.int32)
    p2_ref[...] = jnp.sum(jnp.where(oh2, posv, 0.0), axis=-1,
                          keepdims=True).astype(jnp.int32)

    i8 = (r8 == c8).astype(jnp.float32)
    starts_c = lax.dot_general(i8, starts, (((1,), (1,)), ((), ())),
                               preferred_element_type=jnp.float32)
    blk0 = (lax.broadcasted_iota(jnp.int32, (E, NBLK), 1)
            * BLK).astype(jnp.float32)
    ge = (blk0 >= starts_c).astype(jnp.int32)
    bex_ref[...] = jnp.sum(ge, axis=0, keepdims=True) - 1

    gate_ref[...] = jax.nn.sigmoid(
        jnp.dot(x, sgw1_ref[...], preferred_element_type=jnp.float32))


def _scatter_sc(x_hbm, p1_hbm, p2_hbm, sorted_hbm, rows_v, i1_v, i2_v, sem):
    wid = lax.axis_index("s") * NC + lax.axis_index("c")
    for r in range(TPW // RND):
        base = wid * TPW + r * RND
        pltpu.sync_copy(x_hbm.at[pl.ds(base, RND)], rows_v)
        pltpu.sync_copy(p1_hbm.at[wid, pl.ds(r * RND, RND)], i1_v)
        pltpu.sync_copy(p2_hbm.at[wid, pl.ds(r * RND, RND)], i2_v)
        pltpu.async_copy(rows_v, sorted_hbm.at[i1_v], sem).wait()
        pltpu.async_copy(rows_v, sorted_hbm.at[i2_v], sem).wait()


def _hup_body(bex_ref, xs_ref, gw_ref, uw_ref, h_ref):
    xt = xs_ref[...]
    g = jnp.dot(xt, gw_ref[0], preferred_element_type=jnp.float32)
    u = jnp.dot(xt, uw_ref[0], preferred_element_type=jnp.float32)
    h_ref[...] = (g * jax.nn.sigmoid(g)) * u


def _down_body(bex_ref, h_ref, dw_ref, out_ref):
    out_ref[...] = jnp.dot(h_ref[...], dw_ref[0],
                           preferred_element_type=jnp.float32)


def _gather_sc(dout_hbm, p1_hbm, p2_hbm, g1_hbm, g2_hbm,
               r1_v, r2_v, i1_v, i2_v, sem):
    wid = lax.axis_index("s") * NC + lax.axis_index("c")
    for c in range(TPW // CHUNK):
        base = wid * TPW + c * CHUNK
        pltpu.sync_copy(p1_hbm.at[wid, pl.ds(c * CHUNK, CHUNK)], i1_v)
        pltpu.sync_copy(p2_hbm.at[wid, pl.ds(c * CHUNK, CHUNK)], i2_v)
        cp1 = pltpu.async_copy(dout_hbm.at[i1_v], r1_v, sem)
        cp2 = pltpu.async_copy(dout_hbm.at[i2_v], r2_v, sem)
        cp1.wait()
        cp2.wait()
        pltpu.sync_copy(r1_v, g1_hbm.at[pl.ds(base, CHUNK)])
        pltpu.sync_copy(r2_v, g2_hbm.at[pl.ds(base, CHUNK)])


def _shared_body(x_ref, gw_ref, uw_ref, dw_ref, gate_ref, out_ref, *,
                 bt, ns):
    s = pl.program_id(0)
    t = pl.program_id(1)
    rows = pl.ds(t * bt, bt)
    xt = x_ref[...]
    g = jnp.dot(xt, gw_ref[...], preferred_element_type=jnp.float32)
    u = jnp.dot(xt, uw_ref[...], preferred_element_type=jnp.float32)
    h = (g * jax.nn.sigmoid(g)) * u
    part = jnp.dot(h, dw_ref[...], preferred_element_type=jnp.float32)

    @pl.when(s == 0)
    def _init():
        out_ref[rows, :] = part

    @pl.when(s != 0)
    def _acc():
        out_ref[rows, :] += part

    @pl.when(s == ns - 1)
    def _finalize():
        out_ref[rows, :] = gate_ref[...] * out_ref[rows, :]


def _final_body(g1_ref, g2_ref, shg_ref, w1_ref, w2_ref, out_ref):
    out_ref[...] = (w1_ref[...] * g1_ref[...] + w2_ref[...] * g2_ref[...]
                    + shg_ref[...]) * FINAL_SCALE


def kernel(hidden_states, expert_gate_w, expert_up_w, expert_down_w,
           shared_gate_w, shared_up_w, shared_down_w, router_w,
           shared_expert_gate_w):
    x = hidden_states.reshape(T, D)

    w1, w2, p1, p2, bex, gate = pl.pallas_call(
        _router_body,
        out_shape=(
            jax.ShapeDtypeStruct((T, 1), jnp.float32),
            jax.ShapeDtypeStruct((T, 1), jnp.float32),
            jax.ShapeDtypeStruct((T, 1), jnp.int32),
            jax.ShapeDtypeStruct((T, 1), jnp.int32),
            jax.ShapeDtypeStruct((1, NBLK), jnp.int32),
            jax.ShapeDtypeStruct((T, 1), jnp.float32),
        ),
    )(x, router_w, shared_expert_gate_w)

    p1w = p1.reshape(NW, TPW)
    p2w = p2.reshape(NW, TPW)
    bexf = bex.reshape(NBLK)

    BT = 512
    BS = 512
    NS = DSH // BS
    shg = pl.pallas_call(
        functools.partial(_shared_body, bt=BT, ns=NS),
        grid=(NS, T // BT),
        in_specs=[
            pl.BlockSpec((BT, D), lambda s, t: (t, 0)),
            pl.BlockSpec((D, BS), lambda s, t: (0, s)),
            pl.BlockSpec((D, BS), lambda s, t: (0, s)),
            pl.BlockSpec((BS, D), lambda s, t: (s, 0)),
            pl.BlockSpec((BT, 1), lambda s, t: (t, 0)),
        ],
        out_specs=pl.BlockSpec((T, D), lambda s, t: (0, 0)),
        out_shape=jax.ShapeDtypeStruct((T, D), jnp.float32),
    )(x, shared_gate_w, shared_up_w, shared_down_w, gate)

    mesh = plsc.VectorSubcoreMesh(core_axis_name="c", subcore_axis_name="s")
    sorted_x = pl.kernel(
        _scatter_sc,
        mesh=mesh,
        out_type=jax.ShapeDtypeStruct((S, D), jnp.float32),
        scratch_types=[
            pltpu.VMEM((RND, D), jnp.float32),
            pltpu.VMEM((RND,), jnp.int32),
            pltpu.VMEM((RND,), jnp.int32),
            pltpu.SemaphoreType.DMA,
        ],
    )(x, p1w, p2w)

    h = pl.pallas_call(
        _hup_body,
        grid_spec=pltpu.PrefetchScalarGridSpec(
            num_scalar_prefetch=1,
            grid=(NBLK,),
            in_specs=[
                pl.BlockSpec((BLK, D), lambda i, b: (i, 0)),
                pl.BlockSpec((1, D, DFF), lambda i, b: (b[i], 0, 0)),
                pl.BlockSpec((1, D, DFF), lambda i, b: (b[i], 0, 0)),
            ],
            out_specs=pl.BlockSpec((BLK, DFF), lambda i, b: (i, 0)),
        ),
        out_shape=jax.ShapeDtypeStruct((S, DFF), jnp.float32),
    )(bexf, sorted_x, expert_gate_w, expert_up_w)

    dout = pl.pallas_call(
        _down_body,
        grid_spec=pltpu.PrefetchScalarGridSpec(
            num_scalar_prefetch=1,
            grid=(NBLK,),
            in_specs=[
                pl.BlockSpec((BLK, DFF), lambda i, b: (i, 0)),
                pl.BlockSpec((1, DFF, D), lambda i, b: (b[i], 0, 0)),
            ],
            out_specs=pl.BlockSpec((BLK, D), lambda i, b: (i, 0)),
        ),
        out_shape=jax.ShapeDtypeStruct((S, D), jnp.float32),
    )(bexf, h, expert_down_w)

    g1, g2 = pl.kernel(
        _gather_sc,
        mesh=mesh,
        out_type=(
            jax.ShapeDtypeStruct((T, D), jnp.float32),
            jax.ShapeDtypeStruct((T, D), jnp.float32),
        ),
        scratch_types=[
            pltpu.VMEM((CHUNK, D), jnp.float32),
            pltpu.VMEM((CHUNK, D), jnp.float32),
            pltpu.VMEM((CHUNK,), jnp.int32),
            pltpu.VMEM((CHUNK,), jnp.int32),
            pltpu.SemaphoreType.DMA,
        ],
    )(dout, p1w, p2w)

    out = pl.pallas_call(
        _final_body,
        grid=(T // BT,),
        in_specs=[
            pl.BlockSpec((BT, D), lambda t: (t, 0)),
            pl.BlockSpec((BT, D), lambda t: (t, 0)),
            pl.BlockSpec((BT, D), lambda t: (t, 0)),
            pl.BlockSpec((BT, 1), lambda t: (t, 0)),
            pl.BlockSpec((BT, 1), lambda t: (t, 0)),
        ],
        out_specs=pl.BlockSpec((BT, D), lambda t: (t, 0)),
        out_shape=jax.ShapeDtypeStruct((T, D), jnp.float32),
    )(g1, g2, shg, w1, w2)

    return out


# speedup vs baseline: 1.1662x; 1.0073x over previous
import functools
import math

import jax
import jax.numpy as jnp
from jax import lax
from jax.experimental import pallas as pl
from jax.experimental.pallas import tpu as pltpu
from jax.experimental.pallas import tpu_sc as plsc

E = 8
TOPK = 2
D = 2048
DFF = 1408
DSH = 5632
T = 2048

FINAL_SCALE = 1.0 / math.sqrt(TOPK)

BLK = 256
S = T * TOPK + E * BLK
NBLK = S // BLK

NC = 2
NSUB = 16
NW = NC * NSUB
TPW = T // NW
RND = 32
CHUNK = 16


def _router_body(x_ref, rw_ref, sgw1_ref, w1_ref, w2_ref, p1_ref, p2_ref,
                 bex_ref, gate_ref):
    x = x_ref[...]
    logits = jnp.dot(x, rw_ref[...], preferred_element_type=jnp.float32)
    col = lax.broadcasted_iota(jnp.int32, (T, E), 1)
    m1 = jnp.max(logits, axis=-1, keepdims=True)
    i1 = jnp.min(jnp.where(logits == m1, col, E), axis=-1, keepdims=True)
    oh1 = col == i1
    l2 = jnp.where(oh1, -jnp.inf, logits)
    m2 = jnp.max(l2, axis=-1, keepdims=True)
    i2 = jnp.min(jnp.where(l2 == m2, col, E), axis=-1, keepdims=True)
    oh2 = col == i2
    w1 = 1.0 / (1.0 + jnp.exp(m2 - m1))
    w1_ref[...] = w1
    w2_ref[...] = 1.0 - w1

    m = oh1.astype(jnp.float32) + oh2.astype(jnp.float32)
    counts = jnp.sum(m, axis=0, keepdims=True)
    counts_r = jnp.floor((counts + (BLK - 1)) / BLK) * BLK
    r8 = lax.broadcasted_iota(jnp.int32, (E, E), 0)
    c8 = lax.broadcasted_iota(jnp.int32, (E, E), 1)
    u8 = (r8 < c8).astype(jnp.float32)
    starts = jnp.dot(counts_r, u8, preferred_element_type=jnp.float32)

    rt = lax.broadcasted_iota(jnp.int32, (T, T), 0)
    ct = lax.broadcasted_iota(jnp.int32, (T, T), 1)
    ltri = (ct < rt).astype(jnp.float32)
    ranks = jnp.dot(ltri, m, preferred_element_type=jnp.float32)
    posv = starts + ranks
    p1_ref[...] = jnp.sum(jnp.where(oh1, posv, 0.0), axis=-1,
                          keepdims=True).astype(jnp.int32)
    p2_ref[...] = jnp.sum(jnp.where(oh2, posv, 0.0), axis=-1,
                          keepdims=True).astype(jnp.int32)

    i8 = (r8 == c8).astype(jnp.float32)
    starts_c = lax.dot_general(i8, starts, (((1,), (1,)), ((), ())),
                               preferred_element_type=jnp.float32)
    blk0 = (lax.broadcasted_iota(jnp.int32, (E, NBLK), 1)
            * BLK).astype(jnp.float32)
    ge = (blk0 >= starts_c).astype(jnp.int32)
    bex_ref[...] = jnp.sum(ge, axis=0, keepdims=True) - 1

    gate_ref[...] = jax.nn.sigmoid(
        jnp.dot(x, sgw1_ref[...], preferred_element_type=jnp.float32))


def _scatter_sc(x_hbm, p1_hbm, p2_hbm, sorted_hbm, rows_v, i1_v, i2_v, sem):
    wid = lax.axis_index("s") * NC + lax.axis_index("c")
    for r in range(TPW // RND):
        base = wid * TPW + r * RND
        pltpu.sync_copy(x_hbm.at[pl.ds(base, RND)], rows_v)
        pltpu.sync_copy(p1_hbm.at[wid, pl.ds(r * RND, RND)], i1_v)
        pltpu.sync_copy(p2_hbm.at[wid, pl.ds(r * RND, RND)], i2_v)
        pltpu.async_copy(rows_v, sorted_hbm.at[i1_v], sem).wait()
        pltpu.async_copy(rows_v, sorted_hbm.at[i2_v], sem).wait()


def _hup_body(bex_ref, xs_ref, gw_ref, uw_ref, h_ref):
    xt = xs_ref[...]
    g = jnp.dot(xt, gw_ref[0], preferred_element_type=jnp.float32)
    u = jnp.dot(xt, uw_ref[0], preferred_element_type=jnp.float32)
    h_ref[...] = (g * jax.nn.sigmoid(g)) * u


def _down_body(bex_ref, h_ref, dw_ref, out_ref):
    out_ref[...] = jnp.dot(h_ref[...], dw_ref[0],
                           preferred_element_type=jnp.float32)


def _gather_sc(dout_hbm, p1_hbm, p2_hbm, g1_hbm, g2_hbm,
               r1_v, r2_v, i1_v, i2_v, sem):
    wid = lax.axis_index("s") * NC + lax.axis_index("c")
    for c in range(TPW // CHUNK):
        base = wid * TPW + c * CHUNK
        pltpu.sync_copy(p1_hbm.at[wid, pl.ds(c * CHUNK, CHUNK)], i1_v)
        pltpu.sync_copy(p2_hbm.at[wid, pl.ds(c * CHUNK, CHUNK)], i2_v)
        cp1 = pltpu.async_copy(dout_hbm.at[i1_v], r1_v, sem)
        cp2 = pltpu.async_copy(dout_hbm.at[i2_v], r2_v, sem)
        cp1.wait()
        cp2.wait()
        pltpu.sync_copy(r1_v, g1_hbm.at[pl.ds(base, CHUNK)])
        pltpu.sync_copy(r2_v, g2_hbm.at[pl.ds(base, CHUNK)])


def _shared_body(x_ref, gw_ref, uw_ref, dw_ref, gate_ref, out_ref, *,
                 bt, ns):
    s = pl.program_id(0)
    t = pl.program_id(1)
    rows = pl.ds(t * bt, bt)
    xt = x_ref[...]
    g = jnp.dot(xt, gw_ref[...], preferred_element_type=jnp.float32)
    u = jnp.dot(xt, uw_ref[...], preferred_element_type=jnp.float32)
    h = (g * jax.nn.sigmoid(g)) * u
    part = jnp.dot(h, dw_ref[...], preferred_element_type=jnp.float32)

    @pl.when(s == 0)
    def _init():
        out_ref[rows, :] = part

    @pl.when(s != 0)
    def _acc():
        out_ref[rows, :] += part

    @pl.when(s == ns - 1)
    def _finalize():
        out_ref[rows, :] = gate_ref[...] * out_ref[rows, :]


def _final_body(g1_ref, g2_ref, shg_ref, w1_ref, w2_ref, out_ref):
    out_ref[...] = (w1_ref[...] * g1_ref[...] + w2_ref[...] * g2_ref[...]
                    + shg_ref[...]) * FINAL_SCALE


def kernel(hidden_states, expert_gate_w, expert_up_w, expert_down_w,
           shared_gate_w, shared_up_w, shared_down_w, router_w,
           shared_expert_gate_w):
    x = hidden_states.reshape(T, D)

    w1, w2, p1, p2, bex, gate = pl.pallas_call(
        _router_body,
        out_shape=(
            jax.ShapeDtypeStruct((T, 1), jnp.float32),
            jax.ShapeDtypeStruct((T, 1), jnp.float32),
            jax.ShapeDtypeStruct((T, 1), jnp.int32),
            jax.ShapeDtypeStruct((T, 1), jnp.int32),
            jax.ShapeDtypeStruct((1, NBLK), jnp.int32),
            jax.ShapeDtypeStruct((T, 1), jnp.float32),
        ),
    )(x, router_w, shared_expert_gate_w)

    p1w = p1.reshape(NW, TPW)
    p2w = p2.reshape(NW, TPW)
    bexf = bex.reshape(NBLK)

    BT = 512
    BS = 512
    NS = DSH // BS
    shg = pl.pallas_call(
        functools.partial(_shared_body, bt=BT, ns=NS),
        grid=(NS, T // BT),
        in_specs=[
            pl.BlockSpec((BT, D), lambda s, t: (t, 0)),
            pl.BlockSpec((D, BS), lambda s, t: (0, s)),
            pl.BlockSpec((D, BS), lambda s, t: (0, s)),
            pl.BlockSpec((BS, D), lambda s, t: (s, 0)),
            pl.BlockSpec((BT, 1), lambda s, t: (t, 0)),
        ],
        out_specs=pl.BlockSpec((T, D), lambda s, t: (0, 0)),
        out_shape=jax.ShapeDtypeStruct((T, D), jnp.float32),
    )(x, shared_gate_w, shared_up_w, shared_down_w, gate)

    mesh = plsc.VectorSubcoreMesh(core_axis_name="c", subcore_axis_name="s")
    sorted_x = pl.kernel(
        _scatter_sc,
        mesh=mesh,
        out_type=jax.ShapeDtypeStruct((S, D), jnp.float32),
        scratch_types=[
            pltpu.VMEM((RND, D), jnp.float32),
            pltpu.VMEM((RND,), jnp.int32),
            pltpu.VMEM((RND,), jnp.int32),
            pltpu.SemaphoreType.DMA,
        ],
    )(x, p1w, p2w)

    h = pl.pallas_call(
        _hup_body,
        grid_spec=pltpu.PrefetchScalarGridSpec(
            num_scalar_prefetch=1,
            grid=(NBLK,),
            in_specs=[
                pl.BlockSpec((BLK, D), lambda i, b: (i, 0)),
                pl.BlockSpec((1, D, DFF), lambda i, b: (b[i], 0, 0)),
                pl.BlockSpec((1, D, DFF), lambda i, b: (b[i], 0, 0)),
            ],
            out_specs=pl.BlockSpec((BLK, DFF), lambda i, b: (i, 0)),
        ),
        out_shape=jax.ShapeDtypeStruct((S, DFF), jnp.float32),
    )(bexf, sorted_x, expert_gate_w, expert_up_w)

    dout = pl.pallas_call(
        _down_body,
        grid_spec=pltpu.PrefetchScalarGridSpec(
            num_scalar_prefetch=1,
            grid=(NBLK,),
            in_specs=[
                pl.BlockSpec((BLK, DFF), lambda i, b: (i, 0)),
                pl.BlockSpec((1, DFF, D), lambda i, b: (b[i], 0, 0)),
            ],
            out_specs=pl.BlockSpec((BLK, D), lambda i, b: (i, 0)),
        ),
        out_shape=jax.ShapeDtypeStruct((S, D), jnp.float32),
    )(bexf, h, expert_down_w)

    g1, g2 = pl.kernel(
        _gather_sc,
        mesh=mesh,
        out_type=(
            jax.ShapeDtypeStruct((T, D), jnp.float32),
            jax.ShapeDtypeStruct((T, D), jnp.float32),
        ),
        scratch_types=[
            pltpu.VMEM((CHUNK, D), jnp.float32),
            pltpu.VMEM((CHUNK, D), jnp.float32),
            pltpu.VMEM((CHUNK,), jnp.int32),
            pltpu.VMEM((CHUNK,), jnp.int32),
            pltpu.SemaphoreType.DMA,
        ],
    )(dout, p1w, p2w)

    out = pl.pallas_call(
        _final_body,
        grid=(T // BT,),
        in_specs=[
            pl.BlockSpec((BT, D), lambda t: (t, 0)),
            pl.BlockSpec((BT, D), lambda t: (t, 0)),
            pl.BlockSpec((BT, D), lambda t: (t, 0)),
            pl.BlockSpec((BT, 1), lambda t: (t, 0)),
            pl.BlockSpec((BT, 1), lambda t: (t, 0)),
        ],
        out_specs=pl.BlockSpec((BT, D), lambda t: (t, 0)),
        out_shape=jax.ShapeDtypeStruct((T, D), jnp.float32),
    )(g1, g2, shg, w1, w2)

    return out


# speedup vs baseline: 1.1705x; 1.0036x over previous
import functools
import math

import jax
import jax.numpy as jnp
from jax import lax
from jax.experimental import pallas as pl
from jax.experimental.pallas import tpu as pltpu
from jax.experimental.pallas import tpu_sc as plsc

E = 8
TOPK = 2
D = 2048
DFF = 1408
DSH = 5632
T = 2048

FINAL_SCALE = 1.0 / math.sqrt(TOPK)

BLK = 256
S = T * TOPK + E * BLK
NBLK = S // BLK

NC = 2
NSUB = 16
NW = NC * NSUB
TPW = T // NW
RND = 32
CHUNK = 16


def _router_body(x_ref, rw_ref, sgw1_ref, w1_ref, w2_ref, p1_ref, p2_ref,
                 bex_ref, gate_ref):
    x = x_ref[...]
    logits = jnp.dot(x, rw_ref[...], preferred_element_type=jnp.float32)
    col = lax.broadcasted_iota(jnp.int32, (T, E), 1)
    m1 = jnp.max(logits, axis=-1, keepdims=True)
    i1 = jnp.min(jnp.where(logits == m1, col, E), axis=-1, keepdims=True)
    oh1 = col == i1
    l2 = jnp.where(oh1, -jnp.inf, logits)
    m2 = jnp.max(l2, axis=-1, keepdims=True)
    i2 = jnp.min(jnp.where(l2 == m2, col, E), axis=-1, keepdims=True)
    oh2 = col == i2
    w1 = 1.0 / (1.0 + jnp.exp(m2 - m1))
    w1_ref[...] = w1
    w2_ref[...] = 1.0 - w1

    m = oh1.astype(jnp.float32) + oh2.astype(jnp.float32)
    counts = jnp.sum(m, axis=0, keepdims=True)
    counts_r = jnp.floor((counts + (BLK - 1)) / BLK) * BLK
    r8 = lax.broadcasted_iota(jnp.int32, (E, E), 0)
    c8 = lax.broadcasted_iota(jnp.int32, (E, E), 1)
    u8 = (r8 < c8).astype(jnp.float32)
    starts = jnp.dot(counts_r, u8, preferred_element_type=jnp.float32)

    rt = lax.broadcasted_iota(jnp.int32, (T, T), 0)
    ct = lax.broadcasted_iota(jnp.int32, (T, T), 1)
    ltri = (ct < rt).astype(jnp.float32)
    ranks = jnp.dot(ltri, m, preferred_element_type=jnp.float32)
    posv = starts + ranks
    p1_ref[...] = jnp.sum(jnp.where(oh1, posv, 0.0), axis=-1,
                          keepdims=True).astype(jnp.int32)
    p2_ref[...] = jnp.sum(jnp.where(oh2, posv, 0.0), axis=-1,
                          keepdims=True).astype(jnp.int32)

    i8 = (r8 == c8).astype(jnp.float32)
    starts_c = lax.dot_general(i8, starts, (((1,), (1,)), ((), ())),
                               preferred_element_type=jnp.float32)
    blk0 = (lax.broadcasted_iota(jnp.int32, (E, NBLK), 1)
            * BLK).astype(jnp.float32)
    ge = (blk0 >= starts_c).astype(jnp.int32)
    bex_ref[...] = jnp.sum(ge, axis=0, keepdims=True) - 1

    gate_ref[...] = jax.nn.sigmoid(
        jnp.dot(x, sgw1_ref[...], preferred_element_type=jnp.float32))


def _scatter_sc(x_hbm, p1_hbm, p2_hbm, sorted_hbm, rows_v, i1_v, i2_v, sem):
    wid = lax.axis_index("s") * NC + lax.axis_index("c")
    for r in range(TPW // RND):
        base = wid * TPW + r * RND
        pltpu.sync_copy(x_hbm.at[pl.ds(base, RND)], rows_v)
        pltpu.sync_copy(p1_hbm.at[wid, pl.ds(r * RND, RND)], i1_v)
        pltpu.sync_copy(p2_hbm.at[wid, pl.ds(r * RND, RND)], i2_v)
        pltpu.async_copy(rows_v, sorted_hbm.at[i1_v], sem).wait()
        pltpu.async_copy(rows_v, sorted_hbm.at[i2_v], sem).wait()


def _hup_body(bex_ref, xs_ref, gw_ref, uw_ref, h_ref):
    xt = xs_ref[...]
    g = jnp.dot(xt, gw_ref[0], preferred_element_type=jnp.float32)
    u = jnp.dot(xt, uw_ref[0], preferred_element_type=jnp.float32)
    h_ref[...] = ((g * jax.nn.sigmoid(g)) * u).astype(jnp.bfloat16)


def _down_body(bex_ref, h_ref, dw_ref, out_ref):
    out_ref[...] = jnp.dot(h_ref[...].astype(jnp.float32), dw_ref[0],
                           preferred_element_type=jnp.float32)


def _gather_sc(dout_hbm, p1_hbm, p2_hbm, g1_hbm, g2_hbm,
               r1_v, r2_v, i1_v, i2_v, sem):
    wid = lax.axis_index("s") * NC + lax.axis_index("c")
    for c in range(TPW // CHUNK):
        base = wid * TPW + c * CHUNK
        pltpu.sync_copy(p1_hbm.at[wid, pl.ds(c * CHUNK, CHUNK)], i1_v)
        pltpu.sync_copy(p2_hbm.at[wid, pl.ds(c * CHUNK, CHUNK)], i2_v)
        cp1 = pltpu.async_copy(dout_hbm.at[i1_v], r1_v, sem)
        cp2 = pltpu.async_copy(dout_hbm.at[i2_v], r2_v, sem)
        cp1.wait()
        cp2.wait()
        pltpu.sync_copy(r1_v, g1_hbm.at[pl.ds(base, CHUNK)])
        pltpu.sync_copy(r2_v, g2_hbm.at[pl.ds(base, CHUNK)])


def _shared_body(x_ref, gw_ref, uw_ref, dw_ref, gate_ref, out_ref, *,
                 bt, ns):
    s = pl.program_id(0)
    t = pl.program_id(1)
    rows = pl.ds(t * bt, bt)
    xt = x_ref[...]
    g = jnp.dot(xt, gw_ref[...], preferred_element_type=jnp.float32)
    u = jnp.dot(xt, uw_ref[...], preferred_element_type=jnp.float32)
    h = (g * jax.nn.sigmoid(g)) * u
    part = jnp.dot(h, dw_ref[...], preferred_element_type=jnp.float32)

    @pl.when(s == 0)
    def _init():
        out_ref[rows, :] = part

    @pl.when(s != 0)
    def _acc():
        out_ref[rows, :] += part

    @pl.when(s == ns - 1)
    def _finalize():
        out_ref[rows, :] = gate_ref[...] * out_ref[rows, :]


def _final_body(g1_ref, g2_ref, shg_ref, w1_ref, w2_ref, out_ref):
    out_ref[...] = (w1_ref[...] * g1_ref[...] + w2_ref[...] * g2_ref[...]
                    + shg_ref[...]) * FINAL_SCALE


def kernel(hidden_states, expert_gate_w, expert_up_w, expert_down_w,
           shared_gate_w, shared_up_w, shared_down_w, router_w,
           shared_expert_gate_w):
    x = hidden_states.reshape(T, D)

    w1, w2, p1, p2, bex, gate = pl.pallas_call(
        _router_body,
        out_shape=(
            jax.ShapeDtypeStruct((T, 1), jnp.float32),
            jax.ShapeDtypeStruct((T, 1), jnp.float32),
            jax.ShapeDtypeStruct((T, 1), jnp.int32),
            jax.ShapeDtypeStruct((T, 1), jnp.int32),
            jax.ShapeDtypeStruct((1, NBLK), jnp.int32),
            jax.ShapeDtypeStruct((T, 1), jnp.float32),
        ),
    )(x, router_w, shared_expert_gate_w)

    p1w = p1.reshape(NW, TPW)
    p2w = p2.reshape(NW, TPW)
    bexf = bex.reshape(NBLK)

    BT = 512
    BS = 512
    NS = DSH // BS
    shg = pl.pallas_call(
        functools.partial(_shared_body, bt=BT, ns=NS),
        grid=(NS, T // BT),
        in_specs=[
            pl.BlockSpec((BT, D), lambda s, t: (t, 0)),
            pl.BlockSpec((D, BS), lambda s, t: (0, s)),
            pl.BlockSpec((D, BS), lambda s, t: (0, s)),
            pl.BlockSpec((BS, D), lambda s, t: (s, 0)),
            pl.BlockSpec((BT, 1), lambda s, t: (t, 0)),
        ],
        out_specs=pl.BlockSpec((T, D), lambda s, t: (0, 0)),
        out_shape=jax.ShapeDtypeStruct((T, D), jnp.float32),
    )(x, shared_gate_w, shared_up_w, shared_down_w, gate)

    mesh = plsc.VectorSubcoreMesh(core_axis_name="c", subcore_axis_name="s")
    sorted_x = pl.kernel(
        _scatter_sc,
        mesh=mesh,
        out_type=jax.ShapeDtypeStruct((S, D), jnp.float32),
        scratch_types=[
            pltpu.VMEM((RND, D), jnp.float32),
            pltpu.VMEM((RND,), jnp.int32),
            pltpu.VMEM((RND,), jnp.int32),
            pltpu.SemaphoreType.DMA,
        ],
    )(x, p1w, p2w)

    h = pl.pallas_call(
        _hup_body,
        grid_spec=pltpu.PrefetchScalarGridSpec(
            num_scalar_prefetch=1,
            grid=(NBLK,),
            in_specs=[
                pl.BlockSpec((BLK, D), lambda i, b: (i, 0)),
                pl.BlockSpec((1, D, DFF), lambda i, b: (b[i], 0, 0)),
                pl.BlockSpec((1, D, DFF), lambda i, b: (b[i], 0, 0)),
            ],
            out_specs=pl.BlockSpec((BLK, DFF), lambda i, b: (i, 0)),
        ),
        out_shape=jax.ShapeDtypeStruct((S, DFF), jnp.bfloat16),
    )(bexf, sorted_x, expert_gate_w, expert_up_w)

    dout = pl.pallas_call(
        _down_body,
        grid_spec=pltpu.PrefetchScalarGridSpec(
            num_scalar_prefetch=1,
            grid=(NBLK,),
            in_specs=[
                pl.BlockSpec((BLK, DFF), lambda i, b: (i, 0)),
                pl.BlockSpec((1, DFF, D), lambda i, b: (b[i], 0, 0)),
            ],
            out_specs=pl.BlockSpec((BLK, D), lambda i, b: (i, 0)),
        ),
        out_shape=jax.ShapeDtypeStruct((S, D), jnp.float32),
    )(bexf, h, expert_down_w)

    g1, g2 = pl.kernel(
        _gather_sc,
        mesh=mesh,
        out_type=(
            jax.ShapeDtypeStruct((T, D), jnp.float32),
            jax.ShapeDtypeStruct((T, D), jnp.float32),
        ),
        scratch_types=[
            pltpu.VMEM((CHUNK, D), jnp.float32),
            pltpu.VMEM((CHUNK, D), jnp.float32),
            pltpu.VMEM((CHUNK,), jnp.int32),
            pltpu.VMEM((CHUNK,), jnp.int32),
            pltpu.SemaphoreType.DMA,
        ],
    )(dout, p1w, p2w)

    out = pl.pallas_call(
        _final_body,
        grid=(T // BT,),
        in_specs=[
            pl.BlockSpec((BT, D), lambda t: (t, 0)),
            pl.BlockSpec((BT, D), lambda t: (t, 0)),
            pl.BlockSpec((BT, D), lambda t: (t, 0)),
            pl.BlockSpec((BT, 1), lambda t: (t, 0)),
            pl.BlockSpec((BT, 1), lambda t: (t, 0)),
        ],
        out_specs=pl.BlockSpec((BT, D), lambda t: (t, 0)),
        out_shape=jax.ShapeDtypeStruct((T, D), jnp.float32),
    )(g1, g2, shg, w1, w2)

    return out
